# initial kernel scaffold (unmeasured)
import jax
import jax.numpy as jnp
from jax import lax
from jax.experimental import pallas as pl
from jax.experimental.pallas import tpu as pltpu

B, SQ, SKV, H, D = 8, 8, 1024, 16, 128
SCALE = D ** -0.5


def _partial_body(q_ref, k_ref, v_ref, acc_ref, l_ref):
    for b in range(B):
        s = lax.dot_general(
            q_ref[b], k_ref[b],
            dimension_numbers=(((1,), (1,)), ((), ())),
        ) * SCALE
        p = jnp.exp(s)
        l_ref[b, :] = jnp.sum(p, axis=1)
        acc_ref[b] = lax.dot_general(
            p, v_ref[b],
            dimension_numbers=(((1,), (0,)), ((), ())),
        )


def _combine_body(acc_ref, l_ref, out_ref, recv_acc, recv_l,
                  send_sems, recv_sems):
    my_x = lax.axis_index("x")
    my_y = lax.axis_index("y")
    my_z = lax.axis_index("z")
    partner = (1 - my_x, my_y, my_z)

    barrier_sem = pltpu.get_barrier_semaphore()
    pl.semaphore_signal(barrier_sem, inc=1, device_id=partner,
                        device_id_type=pl.DeviceIdType.MESH)
    pl.semaphore_wait(barrier_sem, 1)

    rdma_acc = pltpu.make_async_remote_copy(
        src_ref=acc_ref, dst_ref=recv_acc,
        send_sem=send_sems.at[0], recv_sem=recv_sems.at[0],
        device_id=partner, device_id_type=pl.DeviceIdType.MESH,
    )
    rdma_l = pltpu.make_async_remote_copy(
        src_ref=l_ref, dst_ref=recv_l,
        send_sem=send_sems.at[1], recv_sem=recv_sems.at[1],
        device_id=partner, device_id_type=pl.DeviceIdType.MESH,
    )
    rdma_acc.start()
    rdma_l.start()
    rdma_acc.wait()
    rdma_l.wait()

    l_tot = l_ref[...] + recv_l[...]
    out_ref[...] = (acc_ref[...] + recv_acc[...]) / l_tot[..., None]


def kernel(Q, K, V):
    acc, l = pl.pallas_call(
        _partial_body,
        grid=(H,),
        in_specs=[
            pl.BlockSpec((B, SQ, None, D), lambda h: (0, 0, h, 0)),
            pl.BlockSpec((B, SKV, None, D), lambda h: (0, 0, h, 0)),
            pl.BlockSpec((B, SKV, None, D), lambda h: (0, 0, h, 0)),
        ],
        out_specs=[
            pl.BlockSpec((B, SQ, None, D), lambda h: (0, 0, h, 0)),
            pl.BlockSpec((B, SQ, None), lambda h: (0, 0, h)),
        ],
        out_shape=[
            jax.ShapeDtypeStruct((B, SQ, H, D), jnp.float32),
            jax.ShapeDtypeStruct((B, SQ, H), jnp.float32),
        ],
    )(Q, K, V)

    out = pl.pallas_call(
        _combine_body,
        in_specs=[
            pl.BlockSpec(memory_space=pltpu.VMEM),
            pl.BlockSpec(memory_space=pltpu.VMEM),
        ],
        out_specs=pl.BlockSpec(memory_space=pltpu.VMEM),
        out_shape=jax.ShapeDtypeStruct((B, SQ, H, D), jnp.float32),
        scratch_shapes=[
            pltpu.VMEM((B, SQ, H, D), jnp.float32),
            pltpu.VMEM((B, SQ, H), jnp.float32),
            pltpu.SemaphoreType.DMA((2,)),
            pltpu.SemaphoreType.DMA((2,)),
        ],
        compiler_params=pltpu.CompilerParams(collective_id=0),
    )(acc, l)
    return out


# baseline (device time: 65640 ns/iter reference)
import jax
import jax.numpy as jnp
from jax import lax
from jax.experimental import pallas as pl
from jax.experimental.pallas import tpu as pltpu

B, SQ, SKV, H, D = 8, 8, 1024, 16, 128
SCALE = D ** -0.5
LEN = SKV


def _partial_body(q_hbm, k_hbm, v_hbm, acc_ref, l_ref,
                  qbuf, kbuf, vbuf, qsems, ksems, vsems):
    h = pl.program_id(0)

    def copies(slot, hh):
        return (
            pltpu.make_async_copy(q_hbm.at[:, :, hh, :], qbuf.at[slot],
                                  qsems.at[slot]),
            pltpu.make_async_copy(k_hbm.at[:, :, hh, :], kbuf.at[slot],
                                  ksems.at[slot]),
            pltpu.make_async_copy(v_hbm.at[:, :, hh, :], vbuf.at[slot],
                                  vsems.at[slot]),
        )

    slot = lax.rem(h, 2)

    @pl.when(h == 0)
    def _():
        for c in copies(0, h):
            c.start()

    @pl.when(h + 1 < H)
    def _():
        for c in copies(lax.rem(h + 1, 2), h + 1):
            c.start()

    for c in copies(slot, h):
        c.wait()

    for b in range(B):
        q_b = qbuf[slot, b]
        k_b = kbuf[slot, b]
        v_b = vbuf[slot, b]
        s = lax.dot_general(
            q_b, k_b, dimension_numbers=(((1,), (1,)), ((), ())),
            preferred_element_type=jnp.float32,
        ) * SCALE
        p = jnp.exp(s)
        l_ref[h, b, :] = jnp.sum(p, axis=1)
        acc_ref[h, b] = lax.dot_general(
            p, v_b, dimension_numbers=(((1,), (0,)), ((), ())),
            preferred_element_type=jnp.float32,
        )


def _combine_body(acc_ref, l_ref, out_ref, recv_acc, recv_l,
                  send_sems, recv_sems):
    my_x = lax.axis_index("x")
    my_y = lax.axis_index("y")
    my_z = lax.axis_index("z")
    partner = (1 - my_x, my_y, my_z)

    rdma_acc = pltpu.make_async_remote_copy(
        src_ref=acc_ref, dst_ref=recv_acc,
        send_sem=send_sems.at[0], recv_sem=recv_sems.at[0],
        device_id=partner, device_id_type=pl.DeviceIdType.MESH,
    )
    rdma_l = pltpu.make_async_remote_copy(
        src_ref=l_ref, dst_ref=recv_l,
        send_sem=send_sems.at[1], recv_sem=recv_sems.at[1],
        device_id=partner, device_id_type=pl.DeviceIdType.MESH,
    )
    rdma_acc.start()
    rdma_l.start()
    rdma_acc.wait()
    rdma_l.wait()

    for h in range(H):
        l_tot = l_ref[h] + recv_l[h]
        o_h = (acc_ref[h] + recv_acc[h]) / l_tot[:, :, None]
        out_ref[:, :, h, :] = o_h


def kernel(Q, K, V):
    acc, l = pl.pallas_call(
        _partial_body,
        grid=(H,),
        in_specs=[
            pl.BlockSpec(memory_space=pl.ANY),
            pl.BlockSpec(memory_space=pl.ANY),
            pl.BlockSpec(memory_space=pl.ANY),
        ],
        out_specs=[
            pl.BlockSpec(memory_space=pltpu.VMEM),
            pl.BlockSpec(memory_space=pltpu.VMEM),
        ],
        out_shape=[
            jax.ShapeDtypeStruct((H, B, SQ, D), jnp.float32),
            jax.ShapeDtypeStruct((H, B, SQ), jnp.float32),
        ],
        scratch_shapes=[
            pltpu.VMEM((2, B, SQ, D), jnp.float32),
            pltpu.VMEM((2, B, LEN, D), jnp.float32),
            pltpu.VMEM((2, B, LEN, D), jnp.float32),
            pltpu.SemaphoreType.DMA((2,)),
            pltpu.SemaphoreType.DMA((2,)),
            pltpu.SemaphoreType.DMA((2,)),
        ],
    )(Q, K, V)

    out = pl.pallas_call(
        _combine_body,
        in_specs=[
            pl.BlockSpec(memory_space=pltpu.VMEM),
            pl.BlockSpec(memory_space=pltpu.VMEM),
        ],
        out_specs=pl.BlockSpec(memory_space=pltpu.VMEM),
        out_shape=jax.ShapeDtypeStruct((B, SQ, H, D), jnp.float32),
        scratch_shapes=[
            pltpu.VMEM((H, B, SQ, D), jnp.float32),
            pltpu.VMEM((H, B, SQ), jnp.float32),
            pltpu.SemaphoreType.DMA((2,)),
            pltpu.SemaphoreType.DMA((2,)),
        ],
    )(acc, l)
    return out
